# baseline (speedup 1.0000x reference)
import functools

import jax
import jax.numpy as jnp
from jax import lax
from jax.experimental import pallas as pl
from jax.experimental.pallas import tpu as pltpu
from jax.experimental.pallas import tpu_sc as plsc

N_DET = 5000
N_GT = 1000
N_THR = 10
N_REC = 101
N_CLS = 20

_TAB_D = 128
_B_PAD = 5120
_NC, _NS = 2, 16
_NW = _NC * _NS
_BD = 1000
_NB = N_DET // _BD
_NCH = 28
_W2 = _NCH * 128
_D2 = 5376
_BG = 672
_NBG = _D2 // _BG


def _sc_gather(tab, idx, n_out):
    bpw = n_out // _NW
    mesh = plsc.VectorSubcoreMesh(core_axis_name="c", subcore_axis_name="s")

    @functools.partial(
        pl.kernel, mesh=mesh,
        out_type=jax.ShapeDtypeStruct((n_out, _TAB_D), jnp.float32),
        scratch_types=[
            pltpu.VMEM((bpw,), jnp.int32),
            pltpu.VMEM((bpw, _TAB_D), jnp.float32),
            pltpu.SemaphoreType.DMA,
        ],
    )
    def gk(tab_hbm, idx_hbm, out_hbm, idx_v, rows_v, sem):
        wid = lax.axis_index("s") * _NC + lax.axis_index("c")
        base = wid * bpw
        pltpu.sync_copy(idx_hbm.at[pl.ds(base, bpw)], idx_v)
        pltpu.async_copy(tab_hbm.at[idx_v], rows_v, sem).wait()
        pltpu.sync_copy(rows_v, out_hbm.at[pl.ds(base, bpw)])

    return gk(tab, idx)


def _iou_block(d, g):
    p = None
    vd = None
    vg = None
    for a in range(3):
        dlo = jnp.min(d[:, 8 * a:8 * a + 8], axis=1, keepdims=True)
        dhi = jnp.max(d[:, 8 * a:8 * a + 8], axis=1, keepdims=True)
        glo = jnp.min(g[8 * a:8 * a + 8, :], axis=0, keepdims=True)
        ghi = jnp.max(g[8 * a:8 * a + 8, :], axis=0, keepdims=True)
        e = jnp.maximum(jnp.minimum(dhi, ghi) - jnp.maximum(dlo, glo), 0.0)
        p = e if p is None else p * e
        vd = (dhi - dlo) if vd is None else vd * (dhi - dlo)
        vg = (ghi - glo) if vg is None else vg * (ghi - glo)
    union = jnp.maximum(vd + vg - p, 1e-9)
    same = d[:, 24:25] == g[24:25, :]
    return jnp.where(same, p / union, 0.0)


def _tc_a_body(det_ref, gt_ref, iou_ref):
    iou_ref[...] = _iou_block(det_ref[...], gt_ref[...])


def _tc_g_body(detg_ref, gt2_ref, csc_ref, kkc_ref, dsg_ref, deg_ref,
               posp_ref, thr_ref, mpack_ref, g2_s, used_s):
    pid = pl.program_id(0)

    dg = detg_ref[...]
    for c in range(_NCH):
        g2_s[pl.ds(c * _BG, _BG), :] = _iou_block(
            dg, gt2_ref[:, c * 128:(c + 1) * 128])

    @pl.when(pid == 0)
    def _init():
        used_s[...] = jnp.zeros((_NCH * N_THR, 128), jnp.float32)

    thr = thr_ref[...]
    iota = lax.broadcasted_iota(jnp.int32, (N_THR, 128), 1)
    pw = jnp.left_shift(
        1, lax.broadcasted_iota(jnp.int32, (N_THR, 1), 0)).astype(jnp.float32)

    for cls in range(N_CLS):
        c0 = csc_ref[cls]
        kk = kkc_ref[cls]
        rs = jnp.clip(dsg_ref[cls] - pid * _BG, 0, _BG)
        re = jnp.clip(deg_ref[cls] - pid * _BG, 0, _BG)

        @pl.when(kk == 1)
        def _fast(c0=c0, rs=rs, re=re):
            def batch_step(b, uw):
                r0 = b * 8
                w8 = g2_s[pl.ds(c0 * _BG + r0, 8), :]
                for j in range(8):
                    cand = jnp.where(
                        uw > 0.0, -1.0,
                        jnp.broadcast_to(w8[j:j + 1, :], (N_THR, 128)))
                    mb = cand
                    for k in (1, 2, 4, 8, 16, 32, 64):
                        mb = jnp.maximum(mb, pltpu.roll(mb, k, 1))
                    okb = mb >= thr
                    fb = jnp.where(cand == mb, iota, 128)
                    for k in (1, 2, 4, 8, 16, 32, 64):
                        fb = jnp.minimum(fb, pltpu.roll(fb, k, 1))
                    fb = jnp.where(okb, fb, -1)
                    uw = jnp.where(iota == fb, 1.0, uw)
                    pos = posp_ref[pid * _BG + r0 + j]
                    mpack_ref[pl.ds(pos, 1), :] = jnp.sum(
                        okb[:, 0:1].astype(jnp.float32) * pw,
                        axis=0, keepdims=True)
                return uw

            uw0 = used_s[pl.ds(c0 * N_THR, N_THR), :]
            uw = lax.fori_loop(rs // 8, re // 8, batch_step, uw0)
            used_s[pl.ds(c0 * N_THR, N_THR), :] = uw

        @pl.when(kk > 1)
        def _slow(c0=c0, kk=kk, rs=rs, re=re):
            def row_step(r, uw):
                w0 = g2_s[pl.ds(c0 * _BG + r, 1), :]
                cand0 = jnp.where(uw > 0.0, -1.0,
                                  jnp.broadcast_to(w0, (N_THR, 128)))
                m0 = jnp.max(cand0, axis=1, keepdims=True)
                f0 = jnp.min(jnp.where(cand0 == m0, iota, 128),
                             axis=1, keepdims=True)

                def scan_chunk(c, mf):
                    m_run, first = mf
                    w = g2_s[pl.ds((c0 + c) * _BG + r, 1), :]
                    u = used_s[pl.ds((c0 + c) * N_THR, N_THR), :]
                    candc = jnp.where(u > 0.0, -1.0,
                                      jnp.broadcast_to(w, (N_THR, 128)))
                    m_c = jnp.max(candc, axis=1, keepdims=True)
                    f_c = jnp.min(jnp.where(candc == m_c, iota, 128),
                                  axis=1, keepdims=True) + c * 128
                    better = m_c > m_run
                    return (jnp.maximum(m_run, m_c),
                            jnp.where(better, f_c, first))

                m, first = lax.fori_loop(1, kk, scan_chunk, (m0, f0))
                ok = m >= thr
                first = jnp.where(ok, first, -1)
                uw = jnp.where(iota == first, 1.0, uw)

                def mark_chunk(c, _):
                    u = used_s[pl.ds((c0 + c) * N_THR, N_THR), :]
                    used_s[pl.ds((c0 + c) * N_THR, N_THR), :] = jnp.where(
                        iota + c * 128 == first, 1.0, u)
                    return 0

                lax.fori_loop(1, kk, mark_chunk, 0)

                pos = posp_ref[pid * _BG + r]
                mpack_ref[pl.ds(pos, 1), :] = jnp.sum(
                    ok.astype(jnp.float32) * pw, axis=0, keepdims=True)
                return uw

            uw0 = used_s[pl.ds(c0 * N_THR, N_THR), :]
            uw = lax.fori_loop(rs, re, row_step, uw0)
            used_s[pl.ds(c0 * N_THR, N_THR), :] = uw


def _tc_b_body(mp_ref, rthr_ref, map_ref):
    v = mp_ref[...].astype(jnp.int32)
    tio = lax.broadcasted_iota(jnp.int32, (N_THR, 1), 0)
    mt = (jnp.right_shift(jnp.broadcast_to(v, (N_THR, N_DET)), tio)
          & 1).astype(jnp.float32)
    tp = mt
    k = 1
    while k < N_DET:
        tp = tp + jnp.pad(tp, ((0, 0), (k, 0)))[:, :N_DET]
        k *= 2
    den = (lax.broadcasted_iota(jnp.int32, (1, N_DET), 1) + 1
           ).astype(jnp.float32)
    prec = tp / den
    rec = tp * (1.0 / N_GT)
    total = jnp.zeros((N_THR, 1), jnp.float32)
    for j in range(N_REC):
        rj = rthr_ref[j]
        total = total + jnp.max(jnp.where(rec >= rj, prec, 0.0),
                                axis=1, keepdims=True)
    map_ref[...] = jnp.sum(total).reshape(1, 1) * (1.0 / (N_REC * N_THR))


def _tc_a_call(det, gtT):
    return pl.pallas_call(
        _tc_a_body,
        grid=(_NB,),
        in_specs=[
            pl.BlockSpec((_BD, 32), lambda i: (i, 0)),
            pl.BlockSpec((32, N_GT), lambda i: (0, 0)),
        ],
        out_specs=pl.BlockSpec((_BD, N_GT), lambda i: (i, 0)),
        out_shape=jax.ShapeDtypeStruct((N_DET, N_GT), jnp.float32),
        compiler_params=pltpu.CompilerParams(
            dimension_semantics=("arbitrary",)),
    )(det, gtT)


def _tc_g_call(detg, gt2T, csc, kkc, dsg, deg, posp, thr):
    return pl.pallas_call(
        _tc_g_body,
        grid=(_NBG,),
        in_specs=[
            pl.BlockSpec((_BG, 32), lambda i: (i, 0)),
            pl.BlockSpec((32, _W2), lambda i: (0, 0)),
            pl.BlockSpec(memory_space=pltpu.SMEM),
            pl.BlockSpec(memory_space=pltpu.SMEM),
            pl.BlockSpec(memory_space=pltpu.SMEM),
            pl.BlockSpec(memory_space=pltpu.SMEM),
            pl.BlockSpec(memory_space=pltpu.SMEM),
            pl.BlockSpec((N_THR, 1), lambda i: (0, 0)),
        ],
        out_specs=pl.BlockSpec((_D2, 1), lambda i: (0, 0)),
        out_shape=jax.ShapeDtypeStruct((_D2, 1), jnp.float32),
        scratch_shapes=[
            pltpu.VMEM((_NCH * _BG, 128), jnp.float32),
            pltpu.VMEM((_NCH * N_THR, 128), jnp.float32),
        ],
        compiler_params=pltpu.CompilerParams(
            dimension_semantics=("arbitrary",)),
    )(detg, gt2T, csc, kkc, dsg, deg, posp, thr)


def _tc_b_call(mpT, rthr):
    return pl.pallas_call(
        _tc_b_body,
        in_specs=[
            pl.BlockSpec((1, N_DET), lambda: (0, 0)),
            pl.BlockSpec(memory_space=pltpu.SMEM),
        ],
        out_specs=pl.BlockSpec((1, 1), lambda: (0, 0)),
        out_shape=jax.ShapeDtypeStruct((1, 1), jnp.float32),
    )(mpT, rthr)


def kernel(pred_boxes, pred_scores, pred_labels, gt_boxes, gt_labels):
    order = jnp.argsort(-pred_scores).astype(jnp.int32)

    def pack(boxes, labels, n):
        c = jnp.transpose(boxes, (0, 2, 1)).reshape(n, 24)
        return jnp.concatenate(
            [c, labels.astype(jnp.float32)[:, None],
             jnp.zeros((n, _TAB_D - 25), jnp.float32)], axis=1)

    det_lab = pred_labels[order]
    gperm = jnp.argsort(det_lab, stable=True).astype(jnp.int32)
    order2 = order[gperm]
    labg = det_lab[gperm]

    dcounts = jnp.bincount(pred_labels, length=N_CLS).astype(jnp.int32)
    ds = jnp.concatenate([jnp.zeros((1,), jnp.int32),
                          jnp.cumsum(dcounts)[:-1].astype(jnp.int32)])
    and8 = ((dcounts + 7) // 8) * 8
    dsg = jnp.concatenate([jnp.zeros((1,), jnp.int32),
                           jnp.cumsum(and8)[:-1].astype(jnp.int32)])
    deg = dsg + and8
    padpos = dsg[labg] + (jnp.arange(N_DET, dtype=jnp.int32) - ds[labg])
    order2p = jnp.full((_D2,), N_DET, jnp.int32).at[padpos].set(order2)
    posp = jnp.full((_D2,), _D2 - 1, jnp.int32).at[padpos].set(gperm)

    counts = jnp.bincount(gt_labels, length=N_CLS)
    kkc = jnp.maximum((counts + 127) // 128, 1).astype(jnp.int32)
    a_off = jnp.concatenate([jnp.zeros((1,), jnp.int32),
                             jnp.cumsum(kkc * 128)[:-1].astype(jnp.int32)])
    csc = a_off // 128
    gt_order = jnp.argsort(gt_labels)
    sl = gt_labels[gt_order]
    starts = jnp.searchsorted(sl, jnp.arange(N_CLS))
    slot = a_off[sl] + (jnp.arange(N_GT) - starts[sl])
    idx2 = jnp.full((_W2,), N_GT, jnp.int32).at[slot].set(
        gt_order.astype(jnp.int32))

    tab = pack(pred_boxes, pred_labels, N_DET)
    dummy_det = pack(jnp.zeros((1, 8, 3), jnp.float32),
                     jnp.full((1,), -1, jnp.int32), 1)
    tab_ext = jnp.concatenate([tab, dummy_det], axis=0)
    padi = jnp.zeros((_B_PAD - N_DET,), jnp.int32)
    det = _sc_gather(tab, jnp.concatenate([order, padi]), _B_PAD)[:N_DET, :32]
    detg = _sc_gather(tab_ext, order2p, _D2)[:, :32]
    gtT = pack(gt_boxes, gt_labels, N_GT)[:, :32].T
    gt_tab_ext = jnp.concatenate(
        [pack(gt_boxes, gt_labels, N_GT),
         pack(jnp.zeros((1, 8, 3), jnp.float32),
              jnp.full((1,), -1, jnp.int32), 1)], axis=0)
    gt2T = _sc_gather(gt_tab_ext, idx2, _W2)[:, :32].T

    thr = jnp.linspace(0.05, 0.5, N_THR).astype(jnp.float32).reshape(N_THR, 1)
    rthr = jnp.linspace(0.0, 1.0, N_REC).astype(jnp.float32)

    ious = _tc_a_call(det, gtT)
    mpack = _tc_g_call(detg, gt2T, csc, kkc, dsg, deg, posp, thr)
    mapv = _tc_b_call(mpack[:N_DET].T, rthr)
    return mapv[0, 0], ious


# speedup vs baseline: 3.8778x; 3.8778x over previous
import functools

import jax
import jax.numpy as jnp
from jax import lax
from jax.experimental import pallas as pl
from jax.experimental.pallas import tpu as pltpu
from jax.experimental.pallas import tpu_sc as plsc

N_DET = 5000
N_GT = 1000
N_THR = 10
N_REC = 101

_B_PAD = 5120
_TAB_D = 128
_NC, _NS = 2, 16
_NW = _NC * _NS
_BPW = _B_PAD // _NW
_BD = 1000
_NB = N_DET // _BD


def _sc_gather(tab, idx):
    mesh = plsc.VectorSubcoreMesh(core_axis_name="c", subcore_axis_name="s")

    @functools.partial(
        pl.kernel, mesh=mesh,
        out_type=jax.ShapeDtypeStruct((_B_PAD, _TAB_D), jnp.float32),
        scratch_types=[
            pltpu.VMEM((_BPW,), jnp.int32),
            pltpu.VMEM((_BPW, _TAB_D), jnp.float32),
            pltpu.SemaphoreType.DMA,
        ],
    )
    def gk(tab_hbm, idx_hbm, out_hbm, idx_v, rows_v, sem):
        wid = lax.axis_index("s") * _NC + lax.axis_index("c")
        base = wid * _BPW
        pltpu.sync_copy(idx_hbm.at[pl.ds(base, _BPW)], idx_v)
        pltpu.async_copy(tab_hbm.at[idx_v], rows_v, sem).wait()
        pltpu.sync_copy(rows_v, out_hbm.at[pl.ds(base, _BPW)])

    return gk(tab, idx)


def _tc_body(det_ref, gt_ref, thr_ref, rthr_ref, iou_ref, map_ref,
             used_s, tp_s, ap_s):
    pid = pl.program_id(0)

    d = det_ref[...]
    g = gt_ref[...]
    p = None
    vd = None
    vg = None
    for a in range(3):
        dlo = jnp.min(d[:, 8 * a:8 * a + 8], axis=1, keepdims=True)
        dhi = jnp.max(d[:, 8 * a:8 * a + 8], axis=1, keepdims=True)
        glo = jnp.min(g[8 * a:8 * a + 8, :], axis=0, keepdims=True)
        ghi = jnp.max(g[8 * a:8 * a + 8, :], axis=0, keepdims=True)
        e = jnp.maximum(jnp.minimum(dhi, ghi) - jnp.maximum(dlo, glo), 0.0)
        p = e if p is None else p * e
        vd = (dhi - dlo) if vd is None else vd * (dhi - dlo)
        vg = (ghi - glo) if vg is None else vg * (ghi - glo)
    union = jnp.maximum(vd + vg - p, 1e-9)
    iou = p / union
    same = d[:, 24:25] == g[24:25, :]
    iou_ref[...] = jnp.where(same, iou, 0.0)

    @pl.when(pid == 0)
    def _init():
        used_s[...] = jnp.zeros((N_THR, N_GT), jnp.float32)
        tp_s[...] = jnp.zeros((N_THR, 128), jnp.float32)
        ap_s[...] = jnp.zeros((N_THR, 128), jnp.float32)

    thr = thr_ref[...]
    rthr = rthr_ref[...]
    iota = lax.broadcasted_iota(jnp.int32, (N_THR, N_GT), 1)

    def chunk_step(c, carry):
        used, tp, ap = carry
        blk = iou_ref[pl.ds(c * 8, 8), :]
        for j in range(8):
            cand = jnp.where(used > 0.0, -1.0,
                             jnp.broadcast_to(blk[j:j + 1, :], (N_THR, N_GT)))
            m = jnp.max(cand, axis=1, keepdims=True)
            ok = m >= thr
            first = jnp.min(jnp.where(cand == m, iota, N_GT),
                            axis=1, keepdims=True)
            used = jnp.where((iota == first) & ok, 1.0, used)
            tp = tp + jnp.where(ok, 1.0, 0.0)
            inv = 1.0 / (pid * _BD + c * 8 + j + 1).astype(jnp.float32)
            ap = jnp.maximum(ap,
                             jnp.where(tp * (1.0 / N_GT) >= rthr, tp * inv, 0.0))
        return used, tp, ap

    carry0 = (used_s[...], tp_s[:, 0:1], ap_s[...])
    used, tp, ap = lax.fori_loop(0, _BD // 8, chunk_step, carry0)
    used_s[...] = used
    tp_s[...] = jnp.broadcast_to(tp, (N_THR, 128))
    ap_s[...] = ap

    @pl.when(pid == _NB - 1)
    def _fin():
        aps = jnp.sum(ap, axis=1, keepdims=True) * (1.0 / N_REC)
        map_ref[...] = jnp.sum(aps).reshape(1, 1) * (1.0 / N_THR)


def _tc_call(det, gtT, thr, rthr):
    return pl.pallas_call(
        _tc_body,
        grid=(_NB,),
        in_specs=[
            pl.BlockSpec((_BD, 32), lambda i: (i, 0)),
            pl.BlockSpec((32, N_GT), lambda i: (0, 0)),
            pl.BlockSpec((N_THR, 1), lambda i: (0, 0)),
            pl.BlockSpec((1, 128), lambda i: (0, 0)),
        ],
        out_specs=[
            pl.BlockSpec((_BD, N_GT), lambda i: (i, 0)),
            pl.BlockSpec((1, 1), lambda i: (0, 0)),
        ],
        out_shape=[
            jax.ShapeDtypeStruct((N_DET, N_GT), jnp.float32),
            jax.ShapeDtypeStruct((1, 1), jnp.float32),
        ],
        scratch_shapes=[
            pltpu.VMEM((N_THR, N_GT), jnp.float32),
            pltpu.VMEM((N_THR, 128), jnp.float32),
            pltpu.VMEM((N_THR, 128), jnp.float32),
        ],
        compiler_params=pltpu.CompilerParams(
            dimension_semantics=("arbitrary",)),
    )(det, gtT, thr, rthr)


def kernel(pred_boxes, pred_scores, pred_labels, gt_boxes, gt_labels):
    order = jnp.argsort(-pred_scores).astype(jnp.int32)
    idx = jnp.concatenate(
        [order, jnp.zeros((_B_PAD - N_DET,), jnp.int32)])

    def pack(boxes, labels, n):
        c = jnp.transpose(boxes, (0, 2, 1)).reshape(n, 24)
        return jnp.concatenate(
            [c, labels.astype(jnp.float32)[:, None],
             jnp.zeros((n, _TAB_D - 25), jnp.float32)], axis=1)

    tab = pack(pred_boxes, pred_labels, N_DET)
    det = _sc_gather(tab, idx)[:N_DET, :32]
    gtT = pack(gt_boxes, gt_labels, N_GT)[:, :32].T

    thr = jnp.linspace(0.05, 0.5, N_THR).astype(jnp.float32).reshape(N_THR, 1)
    rthr = jnp.concatenate(
        [jnp.linspace(0.0, 1.0, N_REC).astype(jnp.float32),
         jnp.full((128 - N_REC,), 2.0, jnp.float32)]).reshape(1, 128)

    ious, mapv = _tc_call(det, gtT, thr, rthr)
    return mapv[0, 0], ious
